# Initial kernel scaffold; baseline (speedup 1.0000x reference)
import jax
import jax.numpy as jnp
from jax.experimental import pallas as pl


def kernel(x, features, hashs):
    raise NotImplementedError("write your pallas kernel here")


# speedup vs baseline: 9.0961x; 9.0961x over previous
import functools

import jax
import jax.numpy as jnp
from jax import lax
from jax.experimental import pallas as pl
from jax.experimental.pallas import tpu as pltpu
from jax.experimental.pallas import tpu_sc as plsc

_INPUT_SIZE = 26
_BATCH = 16384
_DIM = 32
_TABLE = 1 << 20
_MASK = _TABLE - 1
_NW = 32
_BPW = _BATCH // _NW
_L = 16
_NCHUNK = _BPW // _L
_GATHER = 128
_NGATHER = _BPW // _GATHER
_RMAGIC = 12582912.0

_mesh = plsc.VectorSubcoreMesh(core_axis_name="c", subcore_axis_name="s")


@functools.partial(
    pl.kernel,
    mesh=_mesh,
    compiler_params=pltpu.CompilerParams(use_tc_tiling_on_sc=False),
    out_type=jax.ShapeDtypeStruct((_BATCH, _DIM), jnp.float32),
    scratch_types=[
        pltpu.VMEM((_INPUT_SIZE, _BPW), jnp.float32),
        pltpu.VMEM((2, _INPUT_SIZE, _L), jnp.int32),
        pltpu.VMEM((_NGATHER, _GATHER), jnp.int32),
        pltpu.VMEM((_BPW, _DIM), jnp.float32),
        pltpu.SemaphoreType.DMA,
    ],
)
def _hash_embed(x_hbm, hb_hbm, feat_hbm, out_hbm,
                xt_v, hb_v, idx_v, rows_v, sem):
    wid = lax.axis_index("s") * 2 + lax.axis_index("c")
    base = wid * _BPW

    pltpu.sync_copy(x_hbm.at[wid], xt_v)
    pltpu.sync_copy(hb_hbm, hb_v)

    dsplat = [hb_v[1, i] - hb_v[0, i] for i in range(_INPUT_SIZE)]
    h0vec = hb_v[0, 0]
    for i in range(1, _INPUT_SIZE):
        h0vec = h0vec + hb_v[0, i]
    zero = jnp.zeros((_L,), jnp.int32)

    for c in range(_NCHUNK):
        acc = h0vec
        for i in range(_INPUT_SIZE):
            xv = xt_v[i, pl.ds(c * _L, _L)]
            acc = acc + jnp.where(xv > 0.5, dsplat[i], zero)
        idx = jnp.bitwise_and(acc, _MASK)
        g, off = divmod(c * _L, _GATHER)
        idx_v[g, pl.ds(off, _L)] = idx

    copies = [
        pltpu.async_copy(feat_hbm.at[idx_v.at[g]],
                         rows_v.at[pl.ds(g * _GATHER, _GATHER)], sem)
        for g in range(_NGATHER)
    ]
    for cp in copies:
        cp.wait()

    def _quant(b, carry):
        for h in range(_DIM // _L):
            v = rows_v[b, pl.ds(h * _L, _L)]
            v = jnp.minimum(jnp.maximum(v, -1.0), 127.0 / 128.0)
            y = v * 128.0
            r = (y + _RMAGIC) - _RMAGIC
            rows_v[b, pl.ds(h * _L, _L)] = r * (1.0 / 128.0)
        return carry

    lax.fori_loop(0, _BPW, _quant, 0)

    pltpu.sync_copy(rows_v, out_hbm.at[pl.ds(base, _BPW)])


def kernel(x, features, hashs):
    xt = x.T.reshape(_INPUT_SIZE, _NW, _BPW).transpose(1, 0, 2)
    hb = jnp.broadcast_to(hashs.T[:, :, None], (2, _INPUT_SIZE, _L))
    return _hash_embed(xt, hb, features)
